# baseline (speedup 1.0000x reference)
import jax, jax.numpy as jnp
from jax.experimental import pallas as pl

N = 10000
E = 320000
T = 8
D = 128
H = 8
C = D // H
NUM_CLASSES = 10
NUM_GRAPHS = 16


def _structural(x_t, ei, W_lin, att_l, att_r, W_res):
    h = (x_t @ W_lin).reshape(N, H, C)
    al = (h * att_l[None]).sum(-1)
    ar = (h * att_r[None]).sum(-1)
    src, dst = ei[0], ei[1]
    alpha = jax.nn.leaky_relu(al[src] + ar[dst], negative_slope=0.2)
    amax = jax.ops.segment_max(jax.lax.stop_gradient(alpha), dst, num_segments=N)
    amax = jnp.where(jnp.isfinite(amax), amax, 0.0)
    ex = jnp.exp(alpha - amax[dst])
    denom = jax.ops.segment_sum(ex, dst, num_segments=N)
    coef = ex / (denom[dst] + 1e-16)
    msg = h[src] * coef[:, :, None]
    out = jax.ops.segment_sum(msg, dst, num_segments=N).reshape(N, D)
    return out + x_t @ W_res


def _temporal(xs, pos_emb, Wq, Wk, Wv, W_ff, b_ff):
    ti = xs + pos_emb[None]
    q = ti @ Wq
    k = ti @ Wk
    v = ti @ Wv
    def sp(t):
        return t.reshape(N, T, H, C).transpose(0, 2, 1, 3)
    qh, kh, vh = sp(q), sp(k), sp(v)
    scores = jnp.einsum('nhtd,nhsd->nhts', qh, kh) / jnp.sqrt(float(T))
    mask = jnp.tril(jnp.ones((T, T), dtype=bool))
    scores = jnp.where(mask[None, None], scores, -1e9)
    attn = jax.nn.softmax(scores, axis=-1)
    out = jnp.einsum('nhts,nhsd->nhtd', attn, vh)
    out = out.transpose(0, 2, 1, 3).reshape(N, T, D)
    out = jax.nn.relu(out @ W_ff + b_ff) + out
    return out + ti


def _mm_kernel(a_ref, b_ref, o_ref):
    o_ref[...] = a_ref[...] @ b_ref[...]


def kernel(x, edge_index, batch, W_lin, att_l, att_r, W_res, pos_emb, Wq, Wk, Wv, W_ff, b_ff, W_cls, b_cls):
    outs = [_structural(x[t], edge_index[t], W_lin, att_l, att_r, W_res) for t in range(T)]
    s = jnp.stack(outs, axis=1)
    temp = _temporal(s, pos_emb, Wq, Wk, Wv, W_ff, b_ff)
    temporal_pool = temp.mean(axis=1)
    cnt = jax.ops.segment_sum(jnp.ones((N,), dtype=jnp.float32), batch, num_segments=NUM_GRAPHS)
    summ = jax.ops.segment_sum(temporal_pool, batch, num_segments=NUM_GRAPHS)
    pooled = summ / jnp.maximum(cnt, 1.0)[:, None]
    pooled_pad = jnp.pad(pooled, ((0, 0), (0, 0)))
    logits = pl.pallas_call(
        _mm_kernel,
        out_shape=jax.ShapeDtypeStruct((NUM_GRAPHS, NUM_CLASSES), jnp.float32),
    )(pooled_pad, W_cls) + b_cls
    return logits


# speedup vs baseline: 29.4354x; 29.4354x over previous
import functools
import math

import jax
import jax.numpy as jnp
from jax import lax
from jax.experimental import pallas as pl
from jax.experimental.pallas import tpu as pltpu
from jax.experimental.pallas import tpu_sc as plsc

_DEBUG_JNP_EDGE = False
_SPMEM_GATHER = True
_RUN_CHUNKS = False
_DO_ZERO = True
_DO_STAGE = True
_DO_COPYOUT = True

_N = 10000
_E = 320000
_T = 8
_D = 128
_H = 8
_C = _D // _H
_G = 16
_CLS = 10


_BN1 = 1000


def _prep_body(x_ref, wl_ref, attl_ref, attr_ref, h_ref, al_ref, ar_ref, mx_ref):
    i = pl.program_id(1)
    xb = x_ref[0]
    h = xb @ wl_ref[...]
    h_ref[0] = h
    rowh = lax.broadcasted_iota(jnp.int32, (_D, 16), 0) // _C
    colh = lax.broadcasted_iota(jnp.int32, (_D, 16), 1) % _H
    m16 = (rowh == colh).astype(jnp.float32)
    al2 = (h * attl_ref[...]) @ m16
    ar2 = (h * attr_ref[...]) @ m16
    al_ref[0] = al2
    ar_ref[0] = ar2
    mal = jnp.max(al2, axis=0, keepdims=True)
    mar = jnp.max(ar2, axis=0, keepdims=True)
    cur = jnp.concatenate([mal, mar], axis=0)

    @pl.when(i == 0)
    def _():
        mx_ref[0] = cur

    @pl.when(i != 0)
    def _():
        mx_ref[0] = jnp.maximum(mx_ref[0], cur)


def _prep(x, W_lin, att_l, att_r, interpret=False):
    attl = att_l.reshape(1, _D)
    attr = att_r.reshape(1, _D)
    nb = _N // _BN1
    return pl.pallas_call(
        _prep_body,
        grid=(_T, nb),
        in_specs=[
            pl.BlockSpec((1, _BN1, _D), lambda t, i: (t, i, 0)),
            pl.BlockSpec((_D, _D), lambda t, i: (0, 0)),
            pl.BlockSpec((1, _D), lambda t, i: (0, 0)),
            pl.BlockSpec((1, _D), lambda t, i: (0, 0)),
        ],
        out_specs=[
            pl.BlockSpec((1, _BN1, _D), lambda t, i: (t, i, 0)),
            pl.BlockSpec((1, _BN1, 16), lambda t, i: (t, i, 0)),
            pl.BlockSpec((1, _BN1, 16), lambda t, i: (t, i, 0)),
            pl.BlockSpec((1, 2, 16), lambda t, i: (t, 0, 0)),
        ],
        out_shape=[
            jax.ShapeDtypeStruct((_T, _N, _D), jnp.float32),
            jax.ShapeDtypeStruct((_T, _N, 16), jnp.float32),
            jax.ShapeDtypeStruct((_T, _N, 16), jnp.float32),
            jax.ShapeDtypeStruct((_T, 2, 16), jnp.float32),
        ],
        interpret=interpret,
    )(x, W_lin, attl, attr)


_KCH = 40
_NSUB = 16
_EPW = _E // _NSUB
_NCH = _EPW // _KCH
_RPN = _N // _NSUB
_TPC = _T // 2


_ZR = 5


def _sc_edge_body(ei_hbm, ab_hbm, h_hbm, cg_hbm, ex_hbm,
                  msg_hbm, den_hbm,
                  si, di, sg, dg, hb, alb, arb, cgb, ex1d, b1d, ob128,
                  macc, sem):
    core = lax.axis_index("c")
    sub = lax.axis_index("s")
    r0 = sub * _RPN
    z16 = jnp.zeros((16,), jnp.float32)

    def zero_acc():
        for r in range(_ZR):
            for j in range(_H):
                ob128[r, j * 16:(j + 1) * 16] = z16

        def zc(k, c):
            pltpu.sync_copy(ob128, macc.at[pl.ds(r0 + k * _ZR, _ZR)])
            return c
        lax.fori_loop(0, _RPN // _ZR, zc, 0)

    def copy_out(dst_hbm, tn):
        def out(k, c):
            pltpu.sync_copy(macc.at[pl.ds(r0 + k * _ZR, _ZR)], ob128)
            for r in range(_ZR):
                for j in range(_H):
                    b1d[r * _D + j * 16:r * _D + (j + 1) * 16] = \
                        ob128[r, j * 16:(j + 1) * 16]
            pltpu.sync_copy(
                b1d.at[pl.ds(0, _ZR * _D)],
                dst_hbm.at[pl.ds((tn + r0 + k * _ZR) * _D, _ZR * _D)])
            return c
        lax.fori_loop(0, _RPN // _ZR, out, 0)

    def tbody(tt, tcarry):
        t = core * _TPC + tt
        tn = t * _N
        zero_acc()
        pltpu.sync_copy(cg_hbm.at[pl.ds(t * 16, 16)], cgb)
        plsc.subcore_barrier()
        cgv = cgb[...]
        ebase = t * 2 * _E + sub * _EPW
        exbase = (t * _E + sub * _EPW) * 16

        def chunk1(cc, carry):
            base = ebase + cc * _KCH
            pltpu.sync_copy(ei_hbm.at[pl.ds(base, _KCH)], si)
            pltpu.sync_copy(ei_hbm.at[pl.ds(base + _E, _KCH)], di)
            shift = jnp.full((16,), tn, dtype=jnp.int32)
            starts = list(range(0, _KCH - 15, 16))
            if starts[-1] + 16 < _KCH:
                starts.append(_KCH - 16)
            for st in starts:
                sl = pl.ds(st, 16)
                sg[sl] = si[sl] + shift
                dg[sl] = di[sl] + shift
            pltpu.async_copy(h_hbm.at[sg], hb, sem).wait()
            pltpu.async_copy(ab_hbm.at[sg], alb, sem).wait()
            pltpu.async_copy(ab_hbm.at[dg], arb, sem).wait()
            for e in range(_KCH):
                av = alb[e, 0:16]
                bv = arb[e, 16:32]
                s2 = av + bv
                lk = jnp.maximum(s2, s2 * 0.2)
                ex = jnp.exp(lk - cgv)
                ex1d[e * 16:(e + 1) * 16] = ex
                for hh in range(_H):
                    w = ex[hh]
                    csl = slice(hh * _C, hh * _C + _C)
                    hb[e, csl] = hb[e, csl] * w
            pltpu.sync_copy(ex1d, ex_hbm.at[pl.ds(exbase + cc * _KCH * 16,
                                                  _KCH * 16)])
            pltpu.sync_copy(hb, macc.at[di], add=True)
            return carry

        lax.fori_loop(0, _NCH, chunk1, 0)
        plsc.subcore_barrier()
        copy_out(msg_hbm, tn)
        plsc.subcore_barrier()

        zero_acc()
        plsc.subcore_barrier()

        def chunk2(cc, carry):
            base = ebase + cc * _KCH
            pltpu.sync_copy(ei_hbm.at[pl.ds(base + _E, _KCH)], di)
            pltpu.sync_copy(ex_hbm.at[pl.ds(exbase + cc * _KCH * 16,
                                            _KCH * 16)], ex1d)
            for e in range(_KCH):
                exv = ex1d[e * 16:(e + 1) * 16]
                for hh in range(_H):
                    w = exv[hh]
                    csl = slice(hh * _C, hh * _C + _C)
                    alb[e, csl] = jnp.full((16,), w)
            pltpu.sync_copy(alb, macc.at[di], add=True)
            return carry

        lax.fori_loop(0, _NCH, chunk2, 0)
        plsc.subcore_barrier()
        copy_out(den_hbm, tn)
        plsc.subcore_barrier()
        return tcarry

    lax.fori_loop(0, _TPC, tbody, 0)


def _sc_edge(eiflat, ab128, h2, cgflat, interpret=False):
    mesh = plsc.VectorSubcoreMesh(core_axis_name="c", subcore_axis_name="s")
    k = pl.kernel(
        _sc_edge_body,
        out_type=[
            jax.ShapeDtypeStruct((_T * _E * 16,), jnp.float32),
            jax.ShapeDtypeStruct((_T * _N * _D,), jnp.float32),
            jax.ShapeDtypeStruct((_T * _N * _D,), jnp.float32),
        ],
        mesh=mesh,
        scratch_types=[
            pltpu.VMEM((_KCH,), jnp.int32),
            pltpu.VMEM((_KCH,), jnp.int32),
            pltpu.VMEM((_KCH,), jnp.int32),
            pltpu.VMEM((_KCH,), jnp.int32),
            pltpu.VMEM((_KCH, _D), jnp.float32),
            pltpu.VMEM((_KCH, _D), jnp.float32),
            pltpu.VMEM((_KCH, _D), jnp.float32),
            pltpu.VMEM((16,), jnp.float32),
            pltpu.VMEM((_KCH * 16,), jnp.float32),
            pltpu.VMEM((_ZR * _D,), jnp.float32),
            pltpu.VMEM((_ZR, _D), jnp.float32),
            pltpu.VMEM_SHARED((_N, _D), jnp.float32),
            pltpu.SemaphoreType.DMA,
        ],
        compiler_params=pltpu.CompilerParams(needs_layout_passes=False),
        interpret=interpret,
    )
    _, msg, den = k(eiflat, ab128, h2, cgflat)
    return msg, den


_BN3 = 400


def _temp_body(msg_ref, den_ref, x_ref, b_ref, wres_ref, pos_ref, wq_ref,
               wk_ref, wv_ref, wff_ref, bff_ref, wcls_ref, bcls_ref,
               out_ref, pacc, cacc):
    i = pl.program_id(0)
    nb = pl.num_programs(0)
    rh = lax.broadcasted_iota(jnp.int32, (_D, _D), 0) // _C
    chh = lax.broadcasted_iota(jnp.int32, (_D, _D), 1) // _C
    mred = (rh == chh).astype(jnp.float32)

    wres = wres_ref[...]
    ti = []
    for t in range(_T):
        d128 = den_ref[t] + 1e-16
        s_t = msg_ref[t] / d128 + x_ref[t] @ wres
        ti.append(s_t + pos_ref[t:t + 1, :])

    sc = 1.0 / math.sqrt(float(_T))
    qs, ks, vs = [], [], []
    for t in range(_T):
        qs.append((ti[t] @ wq_ref[...]) * sc)
        ks.append(ti[t] @ wk_ref[...])
        vs.append(ti[t] @ wv_ref[...])

    fsum = None
    for t in range(_T):
        ss = [(qs[t] * ks[s]) @ mred for s in range(t + 1)]
        m = ss[0]
        for s in range(1, t + 1):
            m = jnp.maximum(m, ss[s])
        ps = [jnp.exp(v - m) for v in ss]
        dsum = ps[0]
        for s in range(1, t + 1):
            dsum = dsum + ps[s]
        o = ps[0] * vs[0]
        for s in range(1, t + 1):
            o = o + ps[s] * vs[s]
        o = o / dsum
        f = jnp.maximum(o @ wff_ref[...] + bff_ref[...], 0.0) + o + ti[t]
        fsum = f if fsum is None else fsum + f
    tp = fsum * (1.0 / _T)

    bv = b_ref[0, 0]
    ohcol = lax.broadcasted_iota(jnp.int32, (_BN3, _G), 1)
    oh = (bv[:, None] == ohcol).astype(jnp.float32)
    pp = lax.dot_general(oh, tp, (((0,), (0,)), ((), ())))
    cc = lax.dot_general(oh, jnp.ones_like(tp), (((0,), (0,)), ((), ())))

    @pl.when(i == 0)
    def _():
        pacc[...] = pp
        cacc[...] = cc

    @pl.when(i != 0)
    def _():
        pacc[...] = pacc[...] + pp
        cacc[...] = cacc[...] + cc

    @pl.when(i == nb - 1)
    def _():
        pooled = pacc[...] / jnp.maximum(cacc[...], 1.0)
        out_ref[...] = pooled @ wcls_ref[...] + bcls_ref[...]


def _temporal_pool(msg, den, x, batch, W_res, pos_emb, Wq, Wk, Wv, W_ff, b_ff,
                   W_cls, b_cls, interpret=False):
    nb = _N // _BN3
    b3 = batch.reshape(nb, 1, _BN3)
    bff = b_ff.reshape(1, _D)
    bcls = b_cls.reshape(1, _CLS)
    msg4 = msg.reshape(_T, _N, _D)
    den4 = den.reshape(_T, _N, _D)
    full = lambda shp: pl.BlockSpec(shp, lambda i: tuple(0 for _ in shp))
    return pl.pallas_call(
        _temp_body,
        grid=(nb,),
        in_specs=[
            pl.BlockSpec((_T, _BN3, _D), lambda i: (0, i, 0)),
            pl.BlockSpec((_T, _BN3, _D), lambda i: (0, i, 0)),
            pl.BlockSpec((_T, _BN3, _D), lambda i: (0, i, 0)),
            pl.BlockSpec((1, 1, _BN3), lambda i: (i, 0, 0)),
            full((_D, _D)),
            full((_T, _D)),
            full((_D, _D)),
            full((_D, _D)),
            full((_D, _D)),
            full((_D, _D)),
            full((1, _D)),
            full((_D, _CLS)),
            full((1, _CLS)),
        ],
        out_specs=pl.BlockSpec((_G, _CLS), lambda i: (0, 0)),
        out_shape=jax.ShapeDtypeStruct((_G, _CLS), jnp.float32),
        scratch_shapes=[
            pltpu.VMEM((_G, _D), jnp.float32),
            pltpu.VMEM((_G, _D), jnp.float32),
        ],
        interpret=interpret,
    )(msg4, den4, x, b3, W_res, pos_emb, Wq, Wk, Wv, W_ff, bff, W_cls, bcls)


def kernel(x, edge_index, batch, W_lin, att_l, att_r, W_res, pos_emb, Wq, Wk,
           Wv, W_ff, b_ff, W_cls, b_cls):
    h, al2, ar2, mx = _prep(x, W_lin, att_l, att_r)
    s = mx[:, 0, :] + mx[:, 1, :]
    cg2 = jnp.maximum(s, 0.2 * s)
    eiflat = edge_index.reshape(_T * 2 * _E)
    ab128 = jnp.concatenate(
        [al2, ar2, jnp.zeros((_T, _N, _D - 32), jnp.float32)],
        axis=-1).reshape(_T * _N, _D)
    if _DEBUG_JNP_EDGE:
        al = al2[..., :_H]
        ar = ar2[..., :_H]
        src = edge_index[:, 0, :]
        dst = edge_index[:, 1, :]
        msgs, dens = [], []
        for t in range(_T):
            a = jnp.take(al[t], src[t], axis=0) + jnp.take(ar[t], dst[t], axis=0)
            lk = jnp.maximum(a, 0.2 * a)
            ex = jnp.exp(lk - cg2[t, :_H][None])
            dens.append(jax.ops.segment_sum(ex, dst[t], num_segments=_N))
            m = jnp.take(h[t], src[t], axis=0).reshape(_E, _H, _C) * ex[..., None]
            msgs.append(jax.ops.segment_sum(m.reshape(_E, _D), dst[t],
                                            num_segments=_N))
        msg = jnp.stack(msgs).reshape(_T * _N, _D)
        den = jnp.repeat(jnp.stack(dens), _C, axis=-1).reshape(_T * _N, _D)
    else:
        msg, den = _sc_edge(eiflat, ab128, h.reshape(_T * _N, _D),
                            cg2.reshape(_T * 16))
    return _temporal_pool(msg, den, x, batch, W_res, pos_emb, Wq, Wk, Wv,
                          W_ff, b_ff, W_cls, b_cls)


# speedup vs baseline: 45.9589x; 1.5613x over previous
import functools
import math

import jax
import jax.numpy as jnp
from jax import lax
from jax.experimental import pallas as pl
from jax.experimental.pallas import tpu as pltpu
from jax.experimental.pallas import tpu_sc as plsc

_DEBUG_JNP_EDGE = False
_SPMEM_GATHER = True
_RUN_CHUNKS = False
_DO_ZERO = True
_DO_STAGE = True
_DO_COPYOUT = True

_N = 10000
_E = 320000
_T = 8
_D = 128
_H = 8
_C = _D // _H
_G = 16
_CLS = 10


_BN1 = 1000


def _prep_body(x_ref, wl_ref, attl_ref, attr_ref, h_ref, al_ref, ar_ref, mx_ref):
    i = pl.program_id(1)
    xb = x_ref[0]
    h = xb @ wl_ref[...]
    h_ref[0] = h
    rowh = lax.broadcasted_iota(jnp.int32, (_D, 16), 0) // _C
    colh = lax.broadcasted_iota(jnp.int32, (_D, 16), 1) % _H
    m16 = (rowh == colh).astype(jnp.float32)
    al2 = (h * attl_ref[...]) @ m16
    ar2 = (h * attr_ref[...]) @ m16
    al_ref[0] = al2
    ar_ref[0] = ar2
    mal = jnp.max(al2, axis=0, keepdims=True)
    mar = jnp.max(ar2, axis=0, keepdims=True)
    cur = jnp.concatenate([mal, mar], axis=0)

    @pl.when(i == 0)
    def _():
        mx_ref[0] = cur

    @pl.when(i != 0)
    def _():
        mx_ref[0] = jnp.maximum(mx_ref[0], cur)


def _prep(x, W_lin, att_l, att_r, interpret=False):
    attl = att_l.reshape(1, _D)
    attr = att_r.reshape(1, _D)
    nb = _N // _BN1
    return pl.pallas_call(
        _prep_body,
        grid=(_T, nb),
        in_specs=[
            pl.BlockSpec((1, _BN1, _D), lambda t, i: (t, i, 0)),
            pl.BlockSpec((_D, _D), lambda t, i: (0, 0)),
            pl.BlockSpec((1, _D), lambda t, i: (0, 0)),
            pl.BlockSpec((1, _D), lambda t, i: (0, 0)),
        ],
        out_specs=[
            pl.BlockSpec((1, _BN1, _D), lambda t, i: (t, i, 0)),
            pl.BlockSpec((1, _BN1, 16), lambda t, i: (t, i, 0)),
            pl.BlockSpec((1, _BN1, 16), lambda t, i: (t, i, 0)),
            pl.BlockSpec((1, 2, 16), lambda t, i: (t, 0, 0)),
        ],
        out_shape=[
            jax.ShapeDtypeStruct((_T, _N, _D), jnp.float32),
            jax.ShapeDtypeStruct((_T, _N, 16), jnp.float32),
            jax.ShapeDtypeStruct((_T, _N, 16), jnp.float32),
            jax.ShapeDtypeStruct((_T, 2, 16), jnp.float32),
        ],
        interpret=interpret,
    )(x, W_lin, attl, attr)


_KCH = 40
_NSUB = 16
_EPW = _E // _NSUB
_NCH = _EPW // _KCH
_RPN = _N // _NSUB
_TPC = _T // 2


_ZR = 5


def _sc_edge_body(ei_hbm, ab_hbm, h_hbm, cg_hbm, ex_hbm,
                  msg_hbm, den_hbm,
                  si, di, sg, dg, hb, alb, arb, cgb, ex1d, b1d, ob128,
                  macc, sem):
    core = lax.axis_index("c")
    sub = lax.axis_index("s")
    r0 = sub * _RPN
    z16 = jnp.zeros((16,), jnp.float32)

    def zero_acc():
        for r in range(_ZR):
            for j in range(_H):
                ob128[r, j * 16:(j + 1) * 16] = z16

        def zc(k, c):
            pltpu.sync_copy(ob128, macc.at[pl.ds(r0 + k * _ZR, _ZR)])
            return c
        lax.fori_loop(0, _RPN // _ZR, zc, 0)

    def copy_out(dst_hbm, tn):
        def out(k, c):
            pltpu.sync_copy(macc.at[pl.ds(r0 + k * _ZR, _ZR)], ob128)
            for r in range(_ZR):
                for j in range(_H):
                    b1d[r * _D + j * 16:r * _D + (j + 1) * 16] = \
                        ob128[r, j * 16:(j + 1) * 16]
            pltpu.sync_copy(
                b1d.at[pl.ds(0, _ZR * _D)],
                dst_hbm.at[pl.ds((tn + r0 + k * _ZR) * _D, _ZR * _D)])
            return c
        lax.fori_loop(0, _RPN // _ZR, out, 0)

    def tbody(tt, tcarry):
        t = core * _TPC + tt
        tn = t * _N
        zero_acc()
        pltpu.sync_copy(cg_hbm.at[pl.ds(t * 16, 16)], cgb)
        plsc.subcore_barrier()
        cgv = cgb[...]
        ebase = t * 2 * _E + sub * _EPW
        exbase = (t * _E + sub * _EPW) * 16

        def chunk1(cc, carry):
            base = ebase + cc * _KCH
            c1 = pltpu.async_copy(ei_hbm.at[pl.ds(base, _KCH)], si, sem)
            c2 = pltpu.async_copy(ei_hbm.at[pl.ds(base + _E, _KCH)], di, sem)
            c1.wait()
            c2.wait()
            shift = jnp.full((16,), tn, dtype=jnp.int32)
            starts = list(range(0, _KCH - 15, 16))
            if starts[-1] + 16 < _KCH:
                starts.append(_KCH - 16)
            for st in starts:
                sl = pl.ds(st, 16)
                sg[sl] = si[sl] + shift
                dg[sl] = di[sl] + shift
            g1 = pltpu.async_copy(h_hbm.at[sg], hb, sem)
            g2 = pltpu.async_copy(ab_hbm.at[sg], alb, sem)
            g3 = pltpu.async_copy(ab_hbm.at[dg], arb, sem)
            g1.wait()
            g2.wait()
            g3.wait()
            for e in range(_KCH):
                av = alb[e, 0:16]
                bv = arb[e, 16:32]
                s2 = av + bv
                lk = jnp.maximum(s2, s2 * 0.2)
                ex = jnp.exp(lk - cgv)
                ex1d[e * 16:(e + 1) * 16] = ex
                for hh in range(_H):
                    w = ex[hh]
                    csl = slice(hh * _C, hh * _C + _C)
                    hb[e, csl] = hb[e, csl] * w
            pltpu.sync_copy(ex1d, ex_hbm.at[pl.ds(exbase + cc * _KCH * 16,
                                                  _KCH * 16)])
            pltpu.sync_copy(hb, macc.at[di], add=True)
            return carry

        lax.fori_loop(0, _NCH, chunk1, 0)
        plsc.subcore_barrier()
        copy_out(msg_hbm, tn)
        plsc.subcore_barrier()

        zero_acc()
        plsc.subcore_barrier()

        def chunk2(cc, carry):
            base = ebase + cc * _KCH
            c1 = pltpu.async_copy(ei_hbm.at[pl.ds(base + _E, _KCH)], di, sem)
            c2 = pltpu.async_copy(ex_hbm.at[pl.ds(exbase + cc * _KCH * 16,
                                                  _KCH * 16)], ex1d, sem)
            c1.wait()
            c2.wait()
            for e in range(_KCH):
                exv = ex1d[e * 16:(e + 1) * 16]
                for hh in range(_H):
                    w = exv[hh]
                    csl = slice(hh * _C, hh * _C + _C)
                    alb[e, csl] = jnp.full((16,), w)
            pltpu.sync_copy(alb, macc.at[di], add=True)
            return carry

        lax.fori_loop(0, _NCH, chunk2, 0)
        plsc.subcore_barrier()
        copy_out(den_hbm, tn)
        plsc.subcore_barrier()
        return tcarry

    lax.fori_loop(0, _TPC, tbody, 0)


def _sc_edge(eiflat, ab128, h2, cgflat, interpret=False):
    mesh = plsc.VectorSubcoreMesh(core_axis_name="c", subcore_axis_name="s")
    k = pl.kernel(
        _sc_edge_body,
        out_type=[
            jax.ShapeDtypeStruct((_T * _E * 16,), jnp.float32),
            jax.ShapeDtypeStruct((_T * _N * _D,), jnp.float32),
            jax.ShapeDtypeStruct((_T * _N * _D,), jnp.float32),
        ],
        mesh=mesh,
        scratch_types=[
            pltpu.VMEM((_KCH,), jnp.int32),
            pltpu.VMEM((_KCH,), jnp.int32),
            pltpu.VMEM((_KCH,), jnp.int32),
            pltpu.VMEM((_KCH,), jnp.int32),
            pltpu.VMEM((_KCH, _D), jnp.float32),
            pltpu.VMEM((_KCH, _D), jnp.float32),
            pltpu.VMEM((_KCH, _D), jnp.float32),
            pltpu.VMEM((16,), jnp.float32),
            pltpu.VMEM((_KCH * 16,), jnp.float32),
            pltpu.VMEM((_ZR * _D,), jnp.float32),
            pltpu.VMEM((_ZR, _D), jnp.float32),
            pltpu.VMEM_SHARED((_N, _D), jnp.float32),
            pltpu.SemaphoreType.DMA,
        ],
        compiler_params=pltpu.CompilerParams(needs_layout_passes=False),
        interpret=interpret,
    )
    _, msg, den = k(eiflat, ab128, h2, cgflat)
    return msg, den


_BN3 = 400


def _temp_body(msg_ref, den_ref, x_ref, b_ref, wres_ref, pos_ref, wq_ref,
               wk_ref, wv_ref, wff_ref, bff_ref, wcls_ref, bcls_ref,
               out_ref, pacc, cacc):
    i = pl.program_id(0)
    nb = pl.num_programs(0)
    rh = lax.broadcasted_iota(jnp.int32, (_D, _D), 0) // _C
    chh = lax.broadcasted_iota(jnp.int32, (_D, _D), 1) // _C
    mred = (rh == chh).astype(jnp.float32)

    wres = wres_ref[...]
    ti = []
    for t in range(_T):
        d128 = den_ref[t] + 1e-16
        s_t = msg_ref[t] / d128 + x_ref[t] @ wres
        ti.append(s_t + pos_ref[t:t + 1, :])

    sc = 1.0 / math.sqrt(float(_T))
    qs, ks, vs = [], [], []
    for t in range(_T):
        qs.append((ti[t] @ wq_ref[...]) * sc)
        ks.append(ti[t] @ wk_ref[...])
        vs.append(ti[t] @ wv_ref[...])

    fsum = None
    for t in range(_T):
        ss = [(qs[t] * ks[s]) @ mred for s in range(t + 1)]
        m = ss[0]
        for s in range(1, t + 1):
            m = jnp.maximum(m, ss[s])
        ps = [jnp.exp(v - m) for v in ss]
        dsum = ps[0]
        for s in range(1, t + 1):
            dsum = dsum + ps[s]
        o = ps[0] * vs[0]
        for s in range(1, t + 1):
            o = o + ps[s] * vs[s]
        o = o / dsum
        f = jnp.maximum(o @ wff_ref[...] + bff_ref[...], 0.0) + o + ti[t]
        fsum = f if fsum is None else fsum + f
    tp = fsum * (1.0 / _T)

    bv = b_ref[0, 0]
    ohcol = lax.broadcasted_iota(jnp.int32, (_BN3, _G), 1)
    oh = (bv[:, None] == ohcol).astype(jnp.float32)
    pp = lax.dot_general(oh, tp, (((0,), (0,)), ((), ())))
    cc = lax.dot_general(oh, jnp.ones_like(tp), (((0,), (0,)), ((), ())))

    @pl.when(i == 0)
    def _():
        pacc[...] = pp
        cacc[...] = cc

    @pl.when(i != 0)
    def _():
        pacc[...] = pacc[...] + pp
        cacc[...] = cacc[...] + cc

    @pl.when(i == nb - 1)
    def _():
        pooled = pacc[...] / jnp.maximum(cacc[...], 1.0)
        out_ref[...] = pooled @ wcls_ref[...] + bcls_ref[...]


def _temporal_pool(msg, den, x, batch, W_res, pos_emb, Wq, Wk, Wv, W_ff, b_ff,
                   W_cls, b_cls, interpret=False):
    nb = _N // _BN3
    b3 = batch.reshape(nb, 1, _BN3)
    bff = b_ff.reshape(1, _D)
    bcls = b_cls.reshape(1, _CLS)
    msg4 = msg.reshape(_T, _N, _D)
    den4 = den.reshape(_T, _N, _D)
    full = lambda shp: pl.BlockSpec(shp, lambda i: tuple(0 for _ in shp))
    return pl.pallas_call(
        _temp_body,
        grid=(nb,),
        in_specs=[
            pl.BlockSpec((_T, _BN3, _D), lambda i: (0, i, 0)),
            pl.BlockSpec((_T, _BN3, _D), lambda i: (0, i, 0)),
            pl.BlockSpec((_T, _BN3, _D), lambda i: (0, i, 0)),
            pl.BlockSpec((1, 1, _BN3), lambda i: (i, 0, 0)),
            full((_D, _D)),
            full((_T, _D)),
            full((_D, _D)),
            full((_D, _D)),
            full((_D, _D)),
            full((_D, _D)),
            full((1, _D)),
            full((_D, _CLS)),
            full((1, _CLS)),
        ],
        out_specs=pl.BlockSpec((_G, _CLS), lambda i: (0, 0)),
        out_shape=jax.ShapeDtypeStruct((_G, _CLS), jnp.float32),
        scratch_shapes=[
            pltpu.VMEM((_G, _D), jnp.float32),
            pltpu.VMEM((_G, _D), jnp.float32),
        ],
        interpret=interpret,
    )(msg4, den4, x, b3, W_res, pos_emb, Wq, Wk, Wv, W_ff, bff, W_cls, bcls)


def kernel(x, edge_index, batch, W_lin, att_l, att_r, W_res, pos_emb, Wq, Wk,
           Wv, W_ff, b_ff, W_cls, b_cls):
    h, al2, ar2, mx = _prep(x, W_lin, att_l, att_r)
    s = mx[:, 0, :] + mx[:, 1, :]
    cg2 = jnp.maximum(s, 0.2 * s)
    eiflat = edge_index.reshape(_T * 2 * _E)
    ab128 = jnp.concatenate(
        [al2, ar2, jnp.zeros((_T, _N, _D - 32), jnp.float32)],
        axis=-1).reshape(_T * _N, _D)
    if _DEBUG_JNP_EDGE:
        al = al2[..., :_H]
        ar = ar2[..., :_H]
        src = edge_index[:, 0, :]
        dst = edge_index[:, 1, :]
        msgs, dens = [], []
        for t in range(_T):
            a = jnp.take(al[t], src[t], axis=0) + jnp.take(ar[t], dst[t], axis=0)
            lk = jnp.maximum(a, 0.2 * a)
            ex = jnp.exp(lk - cg2[t, :_H][None])
            dens.append(jax.ops.segment_sum(ex, dst[t], num_segments=_N))
            m = jnp.take(h[t], src[t], axis=0).reshape(_E, _H, _C) * ex[..., None]
            msgs.append(jax.ops.segment_sum(m.reshape(_E, _D), dst[t],
                                            num_segments=_N))
        msg = jnp.stack(msgs).reshape(_T * _N, _D)
        den = jnp.repeat(jnp.stack(dens), _C, axis=-1).reshape(_T * _N, _D)
    else:
        msg, den = _sc_edge(eiflat, ab128, h.reshape(_T * _N, _D),
                            cg2.reshape(_T * 16))
    return _temporal_pool(msg, den, x, batch, W_res, pos_emb, Wq, Wk, Wv,
                          W_ff, b_ff, W_cls, b_cls)


# speedup vs baseline: 45.9894x; 1.0007x over previous
import math

import jax
import jax.numpy as jnp
from jax import lax
from jax.experimental import pallas as pl
from jax.experimental.pallas import tpu as pltpu
from jax.experimental.pallas import tpu_sc as plsc

_N = 10000
_E = 320000
_T = 8
_D = 128
_H = 8
_C = _D // _H
_G = 16
_CLS = 10


_BN1 = 1000


def _prep_body(x_ref, wl_ref, attl_ref, attr_ref, h_ref, al_ref, ar_ref, mx_ref):
    i = pl.program_id(1)
    xb = x_ref[0]
    h = xb @ wl_ref[...]
    h_ref[0] = h
    rowh = lax.broadcasted_iota(jnp.int32, (_D, 16), 0) // _C
    colh = lax.broadcasted_iota(jnp.int32, (_D, 16), 1) % _H
    m16 = (rowh == colh).astype(jnp.float32)
    al2 = (h * attl_ref[...]) @ m16
    ar2 = (h * attr_ref[...]) @ m16
    al_ref[0] = al2
    ar_ref[0] = ar2
    mal = jnp.max(al2, axis=0, keepdims=True)
    mar = jnp.max(ar2, axis=0, keepdims=True)
    cur = jnp.concatenate([mal, mar], axis=0)

    @pl.when(i == 0)
    def _():
        mx_ref[0] = cur

    @pl.when(i != 0)
    def _():
        mx_ref[0] = jnp.maximum(mx_ref[0], cur)


def _prep(x, W_lin, att_l, att_r, interpret=False):
    attl = att_l.reshape(1, _D)
    attr = att_r.reshape(1, _D)
    nb = _N // _BN1
    return pl.pallas_call(
        _prep_body,
        grid=(_T, nb),
        in_specs=[
            pl.BlockSpec((1, _BN1, _D), lambda t, i: (t, i, 0)),
            pl.BlockSpec((_D, _D), lambda t, i: (0, 0)),
            pl.BlockSpec((1, _D), lambda t, i: (0, 0)),
            pl.BlockSpec((1, _D), lambda t, i: (0, 0)),
        ],
        out_specs=[
            pl.BlockSpec((1, _BN1, _D), lambda t, i: (t, i, 0)),
            pl.BlockSpec((1, _BN1, 16), lambda t, i: (t, i, 0)),
            pl.BlockSpec((1, _BN1, 16), lambda t, i: (t, i, 0)),
            pl.BlockSpec((1, 2, 16), lambda t, i: (t, 0, 0)),
        ],
        out_shape=[
            jax.ShapeDtypeStruct((_T, _N, _D), jnp.float32),
            jax.ShapeDtypeStruct((_T, _N, 16), jnp.float32),
            jax.ShapeDtypeStruct((_T, _N, 16), jnp.float32),
            jax.ShapeDtypeStruct((_T, 2, 16), jnp.float32),
        ],
        interpret=interpret,
    )(x, W_lin, attl, attr)


_KCH = 40
_NSUB = 16
_EPW = _E // _NSUB
_NCH = _EPW // _KCH
_RPN = _N // _NSUB
_TPC = _T // 2


_ZR = 5


def _sc_edge_body(ei_hbm, ab_hbm, h_hbm, cg_hbm, ex_hbm,
                  msg_hbm, den_hbm,
                  si, di, sg, dg, hb, alb, arb, cgb, ex1d, b1d, ob128,
                  macc, sem):
    core = lax.axis_index("c")
    sub = lax.axis_index("s")
    r0 = sub * _RPN
    z16 = jnp.zeros((16,), jnp.float32)

    def zero_acc():
        for r in range(_ZR):
            for j in range(_H):
                ob128[r, j * 16:(j + 1) * 16] = z16

        def zc(k, c):
            pltpu.sync_copy(ob128, macc.at[pl.ds(r0 + k * _ZR, _ZR)])
            return c
        lax.fori_loop(0, _RPN // _ZR, zc, 0)

    def copy_out(dst_hbm, tn):
        def out(k, c):
            pltpu.sync_copy(macc.at[pl.ds(r0 + k * _ZR, _ZR)], ob128)
            for r in range(_ZR):
                for j in range(_H):
                    b1d[r * _D + j * 16:r * _D + (j + 1) * 16] = \
                        ob128[r, j * 16:(j + 1) * 16]
            pltpu.sync_copy(
                b1d.at[pl.ds(0, _ZR * _D)],
                dst_hbm.at[pl.ds((tn + r0 + k * _ZR) * _D, _ZR * _D)])
            return c
        lax.fori_loop(0, _RPN // _ZR, out, 0)

    def tbody(tt, tcarry):
        t = core * _TPC + tt
        tn = t * _N
        zero_acc()
        pltpu.sync_copy(cg_hbm.at[pl.ds(t * 16, 16)], cgb)
        plsc.subcore_barrier()
        cgv = cgb[...]
        ebase = t * 2 * _E + sub * _EPW
        exbase = (t * _E + sub * _EPW) * 16

        def chunk1(cc, carry):
            base = ebase + cc * _KCH
            c1 = pltpu.async_copy(ei_hbm.at[pl.ds(base, _KCH)], si, sem)
            c2 = pltpu.async_copy(ei_hbm.at[pl.ds(base + _E, _KCH)], di, sem)
            c1.wait()
            c2.wait()
            shift = jnp.full((16,), tn, dtype=jnp.int32)
            starts = list(range(0, _KCH - 15, 16))
            if starts[-1] + 16 < _KCH:
                starts.append(_KCH - 16)
            for st in starts:
                sl = pl.ds(st, 16)
                sg[sl] = si[sl] + shift
                dg[sl] = di[sl] + shift
            g1 = pltpu.async_copy(h_hbm.at[sg], hb, sem)
            g2 = pltpu.async_copy(ab_hbm.at[sg], alb, sem)
            g3 = pltpu.async_copy(ab_hbm.at[dg], arb, sem)
            g1.wait()
            g2.wait()
            g3.wait()
            for e in range(_KCH):
                av = alb[e, 0:16]
                bv = arb[e, 16:32]
                s2 = av + bv
                lk = jnp.maximum(s2, s2 * 0.2)
                ex = jnp.exp(lk - cgv)
                ex1d[e * 16:(e + 1) * 16] = ex
                for hh in range(_H):
                    w = ex[hh]
                    csl = slice(hh * _C, hh * _C + _C)
                    hb[e, csl] = hb[e, csl] * w
            pltpu.sync_copy(ex1d, ex_hbm.at[pl.ds(exbase + cc * _KCH * 16,
                                                  _KCH * 16)])
            pltpu.sync_copy(hb, macc.at[di], add=True)
            return carry

        lax.fori_loop(0, _NCH, chunk1, 0)
        plsc.subcore_barrier()
        copy_out(msg_hbm, tn)
        plsc.subcore_barrier()

        zero_acc()
        plsc.subcore_barrier()

        def chunk2(cc, carry):
            base = ebase + cc * _KCH
            c1 = pltpu.async_copy(ei_hbm.at[pl.ds(base + _E, _KCH)], di, sem)
            c2 = pltpu.async_copy(ex_hbm.at[pl.ds(exbase + cc * _KCH * 16,
                                                  _KCH * 16)], ex1d, sem)
            c1.wait()
            c2.wait()
            for e in range(_KCH):
                exv = ex1d[e * 16:(e + 1) * 16]
                for hh in range(_H):
                    w = exv[hh]
                    csl = slice(hh * _C, hh * _C + _C)
                    alb[e, csl] = jnp.full((16,), w)
            pltpu.sync_copy(alb, macc.at[di], add=True)
            return carry

        lax.fori_loop(0, _NCH, chunk2, 0)
        plsc.subcore_barrier()
        copy_out(den_hbm, tn)
        plsc.subcore_barrier()
        return tcarry

    lax.fori_loop(0, _TPC, tbody, 0)


def _sc_edge(eiflat, ab128, h2, cgflat, interpret=False):
    mesh = plsc.VectorSubcoreMesh(core_axis_name="c", subcore_axis_name="s")
    k = pl.kernel(
        _sc_edge_body,
        out_type=[
            jax.ShapeDtypeStruct((_T * _E * 16,), jnp.float32),
            jax.ShapeDtypeStruct((_T * _N * _D,), jnp.float32),
            jax.ShapeDtypeStruct((_T * _N * _D,), jnp.float32),
        ],
        mesh=mesh,
        scratch_types=[
            pltpu.VMEM((_KCH,), jnp.int32),
            pltpu.VMEM((_KCH,), jnp.int32),
            pltpu.VMEM((_KCH,), jnp.int32),
            pltpu.VMEM((_KCH,), jnp.int32),
            pltpu.VMEM((_KCH, _D), jnp.float32),
            pltpu.VMEM((_KCH, _D), jnp.float32),
            pltpu.VMEM((_KCH, _D), jnp.float32),
            pltpu.VMEM((16,), jnp.float32),
            pltpu.VMEM((_KCH * 16,), jnp.float32),
            pltpu.VMEM((_ZR * _D,), jnp.float32),
            pltpu.VMEM((_ZR, _D), jnp.float32),
            pltpu.VMEM_SHARED((_N, _D), jnp.float32),
            pltpu.SemaphoreType.DMA,
        ],
        compiler_params=pltpu.CompilerParams(needs_layout_passes=False),
        interpret=interpret,
    )
    _, msg, den = k(eiflat, ab128, h2, cgflat)
    return msg, den


_BN3 = 400


def _temp_body(msg_ref, den_ref, x_ref, b_ref, wres_ref, pos_ref, wq_ref,
               wk_ref, wv_ref, wff_ref, bff_ref, wcls_ref, bcls_ref,
               out_ref, pacc, cacc):
    i = pl.program_id(0)
    nb = pl.num_programs(0)
    rh = lax.broadcasted_iota(jnp.int32, (_D, _D), 0) // _C
    chh = lax.broadcasted_iota(jnp.int32, (_D, _D), 1) // _C
    mred = (rh == chh).astype(jnp.float32)

    wres = wres_ref[...]
    ti = []
    for t in range(_T):
        d128 = den_ref[t] + 1e-16
        s_t = msg_ref[t] / d128 + x_ref[t] @ wres
        ti.append(s_t + pos_ref[t:t + 1, :])

    sc = 1.0 / math.sqrt(float(_T))
    qs, ks, vs = [], [], []
    for t in range(_T):
        qs.append((ti[t] @ wq_ref[...]) * sc)
        ks.append(ti[t] @ wk_ref[...])
        vs.append(ti[t] @ wv_ref[...])

    fsum = None
    for t in range(_T):
        ss = [(qs[t] * ks[s]) @ mred for s in range(t + 1)]
        m = ss[0]
        for s in range(1, t + 1):
            m = jnp.maximum(m, ss[s])
        ps = [jnp.exp(v - m) for v in ss]
        dsum = ps[0]
        for s in range(1, t + 1):
            dsum = dsum + ps[s]
        o = ps[0] * vs[0]
        for s in range(1, t + 1):
            o = o + ps[s] * vs[s]
        o = o / dsum
        f = jnp.maximum(o @ wff_ref[...] + bff_ref[...], 0.0) + o + ti[t]
        fsum = f if fsum is None else fsum + f
    tp = fsum * (1.0 / _T)

    bv = b_ref[0, 0]
    ohcol = lax.broadcasted_iota(jnp.int32, (_BN3, _G), 1)
    oh = (bv[:, None] == ohcol).astype(jnp.float32)
    pp = lax.dot_general(oh, tp, (((0,), (0,)), ((), ())))
    cc = lax.dot_general(oh, jnp.ones_like(tp), (((0,), (0,)), ((), ())))

    @pl.when(i == 0)
    def _():
        pacc[...] = pp
        cacc[...] = cc

    @pl.when(i != 0)
    def _():
        pacc[...] = pacc[...] + pp
        cacc[...] = cacc[...] + cc

    @pl.when(i == nb - 1)
    def _():
        pooled = pacc[...] / jnp.maximum(cacc[...], 1.0)
        out_ref[...] = pooled @ wcls_ref[...] + bcls_ref[...]


def _temporal_pool(msg, den, x, batch, W_res, pos_emb, Wq, Wk, Wv, W_ff, b_ff,
                   W_cls, b_cls, interpret=False):
    nb = _N // _BN3
    b3 = batch.reshape(nb, 1, _BN3)
    bff = b_ff.reshape(1, _D)
    bcls = b_cls.reshape(1, _CLS)
    msg4 = msg.reshape(_T, _N, _D)
    den4 = den.reshape(_T, _N, _D)
    full = lambda shp: pl.BlockSpec(shp, lambda i: tuple(0 for _ in shp))
    return pl.pallas_call(
        _temp_body,
        grid=(nb,),
        in_specs=[
            pl.BlockSpec((_T, _BN3, _D), lambda i: (0, i, 0)),
            pl.BlockSpec((_T, _BN3, _D), lambda i: (0, i, 0)),
            pl.BlockSpec((_T, _BN3, _D), lambda i: (0, i, 0)),
            pl.BlockSpec((1, 1, _BN3), lambda i: (i, 0, 0)),
            full((_D, _D)),
            full((_T, _D)),
            full((_D, _D)),
            full((_D, _D)),
            full((_D, _D)),
            full((_D, _D)),
            full((1, _D)),
            full((_D, _CLS)),
            full((1, _CLS)),
        ],
        out_specs=pl.BlockSpec((_G, _CLS), lambda i: (0, 0)),
        out_shape=jax.ShapeDtypeStruct((_G, _CLS), jnp.float32),
        scratch_shapes=[
            pltpu.VMEM((_G, _D), jnp.float32),
            pltpu.VMEM((_G, _D), jnp.float32),
        ],
        interpret=interpret,
    )(msg4, den4, x, b3, W_res, pos_emb, Wq, Wk, Wv, W_ff, bff, W_cls, bcls)


def kernel(x, edge_index, batch, W_lin, att_l, att_r, W_res, pos_emb, Wq, Wk,
           Wv, W_ff, b_ff, W_cls, b_cls):
    h, al2, ar2, mx = _prep(x, W_lin, att_l, att_r)
    s = mx[:, 0, :] + mx[:, 1, :]
    cg2 = jnp.maximum(s, 0.2 * s)
    eiflat = edge_index.reshape(_T * 2 * _E)
    ab128 = jnp.concatenate(
        [al2, ar2, jnp.zeros((_T, _N, _D - 32), jnp.float32)],
        axis=-1).reshape(_T * _N, _D)
    msg, den = _sc_edge(eiflat, ab128, h.reshape(_T * _N, _D),
                        cg2.reshape(_T * 16))
    return _temporal_pool(msg, den, x, batch, W_res, pos_emb, Wq, Wk, Wv,
                          W_ff, b_ff, W_cls, b_cls)
